# baseline (speedup 1.0000x reference)
import functools

import jax
import jax.numpy as jnp
from jax.experimental import pallas as pl
from jax.experimental.pallas import tpu as pltpu

N_E = 64
D_M = 768
D_F = 1024
SEQ = 2048
TOPK = SEQ // N_E


def _moe_body(x_ref, xb_ref, choice_ref, w1_ref, w2_ref, out_ref, probs_ref):
    e = pl.program_id(0)

    @pl.when(e == 0)
    def _compute_probs():
        xt = x_ref[...]
        ch = choice_ref[...]
        logits_t = jax.lax.dot_general(
            ch, xt, (((1,), (1,)), ((), ())),
            preferred_element_type=jnp.float32)
        m = jnp.max(logits_t, axis=0, keepdims=True)
        p = jnp.exp(logits_t - m)
        probs_ref[...] = p / jnp.sum(p, axis=0, keepdims=True)

    row = probs_ref[pl.ds(e, 1), :]
    lane = jax.lax.broadcasted_iota(jnp.int32, (1, SEQ), 1)
    rowk = jax.lax.broadcasted_iota(jnp.int32, (TOPK, 1), 0)

    def topk_step(j, carry):
        r, idxs, gates = carry
        m = jnp.max(r, axis=1, keepdims=True)
        idx = jnp.min(jnp.where(r == m, lane, SEQ), axis=1,
                      keepdims=True)
        r = jnp.where(lane == idx, -1.0, r)
        idxs = jnp.where(rowk == j, idx, idxs)
        gates = jnp.where(rowk == j, m, gates)
        return r, idxs, gates

    idxs0 = jnp.zeros((TOPK, 1), jnp.int32)
    gates0 = jnp.zeros((TOPK, 1), jnp.float32)
    _, idxs, gates = jax.lax.fori_loop(0, TOPK, topk_step,
                                       (row, idxs0, gates0))

    lane_ks = jax.lax.broadcasted_iota(jnp.int32, (TOPK, SEQ), 1)
    P = (lane_ks == idxs).astype(jnp.bfloat16)

    x_g = jax.lax.dot_general(P, xb_ref[...], (((1,), (0,)), ((), ())),
                              preferred_element_type=jnp.float32)
    x_g = x_g.astype(jnp.bfloat16)
    h = jax.lax.dot_general(x_g, w1_ref[0], (((1,), (1,)), ((), ())),
                            preferred_element_type=jnp.float32)
    h = h / (1.0 + jnp.exp(-h))
    y = jax.lax.dot_general(h.astype(jnp.bfloat16), w2_ref[0],
                            (((1,), (1,)), ((), ())),
                            preferred_element_type=jnp.float32)
    y = (y * gates).astype(jnp.bfloat16)

    scat = jax.lax.dot_general(P, y, (((0,), (0,)), ((), ())),
                               preferred_element_type=jnp.float32)

    @pl.when(e == 0)
    def _init():
        out_ref[...] = scat

    @pl.when(e != 0)
    def _acc():
        out_ref[...] += scat


@functools.partial(jax.jit, static_argnames=("interpret",))
def kernel(x, choice, w1, w2, interpret=False):
    x2d = x[0]
    xb = x2d.astype(jnp.bfloat16)
    w1b = w1.astype(jnp.bfloat16)
    w2b = w2.astype(jnp.bfloat16)
    out = pl.pallas_call(
        _moe_body,
        grid=(N_E,),
        in_specs=[
            pl.BlockSpec((SEQ, D_M), lambda e: (0, 0)),
            pl.BlockSpec((SEQ, D_M), lambda e: (0, 0)),
            pl.BlockSpec((N_E, D_M), lambda e: (0, 0)),
            pl.BlockSpec((1, D_F, D_M), lambda e: (e, 0, 0)),
            pl.BlockSpec((1, D_M, D_F), lambda e: (e, 0, 0)),
        ],
        out_specs=pl.BlockSpec((SEQ, D_M), lambda e: (0, 0)),
        out_shape=jax.ShapeDtypeStruct((SEQ, D_M), jnp.float32),
        scratch_shapes=[pltpu.VMEM((N_E, SEQ), jnp.float32)],
        compiler_params=pltpu.CompilerParams(
            dimension_semantics=("arbitrary",)),
        interpret=interpret,
    )(x2d, xb, choice, w1b, w2b)
    return out[None]


# speedup vs baseline: 5.7336x; 5.7336x over previous
import functools

import jax
import jax.numpy as jnp
from jax.experimental import pallas as pl
from jax.experimental.pallas import tpu as pltpu

N_E = 64
D_M = 768
D_F = 1024
SEQ = 2048
TOPK = SEQ // N_E
CHUNK = 8
ROWS = CHUNK * TOPK


def _moe_body(x_ref, choice_ref, w1_ref, w2_ref, out_ref,
              idx_ref, gate_ref, ct_ref, xb_ref, xg_ref, yall_ref):
    e = pl.program_id(0)

    @pl.when(e == 0)
    def _route():
        xt = x_ref[...]
        xb_ref[...] = xt.astype(jnp.bfloat16)
        ch = choice_ref[...]
        logits_t = jax.lax.dot_general(
            ch, xt, (((1,), (1,)), ((), ())),
            preferred_element_type=jnp.float32)
        m = jnp.max(logits_t, axis=0, keepdims=True)
        p = jnp.exp(logits_t - m)
        probs = p / jnp.sum(p, axis=0, keepdims=True)

        lane = jax.lax.broadcasted_iota(jnp.int32, (N_E, SEQ), 1)
        colk = jax.lax.broadcasted_iota(jnp.int32, (N_E, TOPK), 1)

        def topk_step(j, carry):
            r, idxs, gates = carry
            mv = jnp.max(r, axis=1, keepdims=True)
            ix = jnp.min(jnp.where(r == mv, lane, SEQ), axis=1,
                         keepdims=True)
            r = jnp.where(lane == ix, -1.0, r)
            idxs = jnp.where(colk == j, ix, idxs)
            gates = jnp.where(colk == j, mv, gates)
            return r, idxs, gates

        idxs0 = jnp.zeros((N_E, TOPK), jnp.int32)
        gates0 = jnp.zeros((N_E, TOPK), jnp.float32)
        _, idxs, gates = jax.lax.fori_loop(0, TOPK, topk_step,
                                           (probs, idxs0, gates0))
        idx_ref[...] = idxs
        gate_ref[...] = gates

    @pl.when(e < N_E // CHUNK)
    def _gather_chunk():
        lane_ks = jax.lax.broadcasted_iota(jnp.int32, (TOPK, SEQ), 1)
        for i in range(CHUNK):
            ee = e * CHUNK + i
            idxs_e = idx_ref[pl.ds(ee, 1), :]
            idxs_c = jax.lax.broadcast_in_dim(idxs_e[0], (TOPK, 1), (0,))
            strip = (lane_ks == idxs_c).astype(jnp.bfloat16)
            ct_ref[pl.ds(ee * TOPK, TOPK), :] = strip
        xg_ref[pl.ds(e * ROWS, ROWS), :] = jax.lax.dot_general(
            ct_ref[pl.ds(e * ROWS, ROWS), :], xb_ref[...],
            (((1,), (0,)), ((), ())),
            preferred_element_type=jnp.float32)

    gates = gate_ref[pl.ds(e, 1), :]
    gates_c = jax.lax.broadcast_in_dim(gates[0], (TOPK, 1), (0,))

    x_g = xg_ref[pl.ds(e * TOPK, TOPK), :]
    h = jax.lax.dot_general(x_g, w1_ref[0], (((1,), (1,)), ((), ())),
                            preferred_element_type=jnp.float32)
    h = h / (1.0 + jnp.exp(-h))
    y = jax.lax.dot_general(h, w2_ref[0], (((1,), (1,)), ((), ())),
                            preferred_element_type=jnp.float32)
    yall_ref[pl.ds(e * TOPK, TOPK), :] = (y * gates_c).astype(jnp.bfloat16)

    @pl.when(e == N_E - 1)
    def _combine():
        out_ref[...] = jax.lax.dot_general(
            ct_ref[...], yall_ref[...], (((0,), (0,)), ((), ())),
            preferred_element_type=jnp.float32)


@functools.partial(jax.jit, static_argnames=("interpret",))
def kernel(x, choice, w1, w2, interpret=False):
    x2d = x[0]
    out = pl.pallas_call(
        _moe_body,
        grid=(N_E,),
        in_specs=[
            pl.BlockSpec((SEQ, D_M), lambda e: (0, 0)),
            pl.BlockSpec((N_E, D_M), lambda e: (0, 0)),
            pl.BlockSpec((1, D_F, D_M), lambda e: (e, 0, 0)),
            pl.BlockSpec((1, D_M, D_F), lambda e: (e, 0, 0)),
        ],
        out_specs=pl.BlockSpec((SEQ, D_M), lambda e: (0, 0)),
        out_shape=jax.ShapeDtypeStruct((SEQ, D_M), jnp.float32),
        scratch_shapes=[pltpu.VMEM((N_E, TOPK), jnp.int32),
                        pltpu.VMEM((N_E, TOPK), jnp.float32),
                        pltpu.VMEM((SEQ, SEQ), jnp.bfloat16),
                        pltpu.VMEM((SEQ, D_M), jnp.bfloat16),
                        pltpu.VMEM((SEQ, D_M), jnp.float32),
                        pltpu.VMEM((SEQ, D_M), jnp.bfloat16)],
        compiler_params=pltpu.CompilerParams(
            dimension_semantics=("arbitrary",)),
        interpret=interpret,
    )(x2d, choice, w1, w2)
    return out[None]


# speedup vs baseline: 5.8854x; 1.0265x over previous
import jax
import jax.numpy as jnp
from jax.experimental import pallas as pl
from jax.experimental.pallas import tpu as pltpu

N_E = 64
D_M = 768
D_F = 1024
SEQ = 2048
TOPK = SEQ // N_E
CHUNK = 8
ROWS = CHUNK * TOPK


def _cumsum_lanes(x):
    n = x.shape[1]
    lane = jax.lax.broadcasted_iota(jnp.int32, x.shape, 1)
    sh = 1
    while sh < n:
        x = x + jnp.where(lane >= sh, pltpu.roll(x, sh, axis=1), 0.0)
        sh *= 2
    return x


def _moe_body(x_ref, choice_ref, w1_ref, w2_ref, out_ref,
              gm_ref, rank_ref, ct_ref, xb_ref, xg_ref, yall_ref):
    e = pl.program_id(0)

    @pl.when(e == 0)
    def _route():
        xt = x_ref[...]
        xb_ref[...] = xt.astype(jnp.bfloat16)
        ch = choice_ref[...]
        logits_t = jax.lax.dot_general(
            ch, xt, (((1,), (1,)), ((), ())),
            preferred_element_type=jnp.float32)
        m = jnp.max(logits_t, axis=0, keepdims=True)
        p = jnp.exp(logits_t - m)
        probs = p / jnp.sum(p, axis=0, keepdims=True)

        def topk_step(j, r):
            mv = jnp.max(r, axis=1, keepdims=True)
            return jnp.where(r == mv, -1.0, r)

        r_fin = jax.lax.fori_loop(0, TOPK, topk_step, probs)
        selF = jnp.where(r_fin < 0.0, 1.0, 0.0)

        def trim_cond(sF):
            return jnp.max(jnp.sum(sF, axis=1)) > TOPK

        def trim_body(sF):
            cnt = jnp.sum(sF, axis=1, keepdims=True)
            need = cnt > TOPK
            vmin = jnp.min(jnp.where(sF > 0.0, probs, 2.0), axis=1,
                           keepdims=True)
            inclF = jnp.where((sF > 0.0) & (probs == vmin), 1.0, 0.0)
            ncl = jnp.sum(inclF, axis=1, keepdims=True)
            keep = TOPK - (cnt - ncl)
            ccs = _cumsum_lanes(inclF)
            newsF = sF * jnp.where((inclF > 0.0) & (ccs > keep), 0.0, 1.0)
            return jnp.where(need, newsF, sF)

        selF = jax.lax.while_loop(trim_cond, trim_body, selF)
        gm_ref[...] = selF * probs
        rank_ref[...] = (_cumsum_lanes(selF) - selF).astype(jnp.int32)

    @pl.when(e < N_E // CHUNK)
    def _gather_chunk():
        c = e
        rowk = jax.lax.broadcasted_iota(jnp.int32, (TOPK, SEQ), 0)
        strips = []
        for i in range(CHUNK):
            ee = c * CHUNK + i
            gmr = gm_ref[pl.ds(ee, 1), :]
            rkr = rank_ref[pl.ds(ee, 1), :]
            selr = gmr > 0.0
            onehot = (rkr == rowk) & selr
            strips.append(onehot.astype(jnp.bfloat16))
            ct_ref[pl.ds(ee * TOPK, TOPK), :] = jnp.where(
                onehot, gmr, 0.0).astype(jnp.bfloat16)
        pg = jnp.concatenate(strips, axis=0)
        xg_ref[pl.ds(c * ROWS, ROWS), :] = jax.lax.dot_general(
            pg, xb_ref[...], (((1,), (0,)), ((), ())),
            preferred_element_type=jnp.float32)

    x_g = xg_ref[pl.ds(e * TOPK, TOPK), :]
    h = jax.lax.dot_general(x_g, w1_ref[0], (((1,), (1,)), ((), ())),
                            preferred_element_type=jnp.float32)
    h = h / (1.0 + jnp.exp(-h))
    y = jax.lax.dot_general(h, w2_ref[0], (((1,), (1,)), ((), ())),
                            preferred_element_type=jnp.float32)
    yall_ref[pl.ds(e * TOPK, TOPK), :] = y.astype(jnp.bfloat16)

    @pl.when(e == N_E - 1)
    def _combine():
        out_ref[...] = jax.lax.dot_general(
            ct_ref[...], yall_ref[...], (((0,), (0,)), ((), ())),
            preferred_element_type=jnp.float32)


@jax.jit
def kernel(x, choice, w1, w2):
    x2d = x[0]
    out = pl.pallas_call(
        _moe_body,
        grid=(N_E,),
        in_specs=[
            pl.BlockSpec((SEQ, D_M), lambda e: (0, 0)),
            pl.BlockSpec((N_E, D_M), lambda e: (0, 0)),
            pl.BlockSpec((1, D_F, D_M), lambda e: (e, 0, 0)),
            pl.BlockSpec((1, D_M, D_F), lambda e: (e, 0, 0)),
        ],
        out_specs=pl.BlockSpec((SEQ, D_M), lambda e: (0, 0)),
        out_shape=jax.ShapeDtypeStruct((SEQ, D_M), jnp.float32),
        scratch_shapes=[pltpu.VMEM((N_E, SEQ), jnp.float32),
                        pltpu.VMEM((N_E, SEQ), jnp.int32),
                        pltpu.VMEM((SEQ, SEQ), jnp.bfloat16),
                        pltpu.VMEM((SEQ, D_M), jnp.bfloat16),
                        pltpu.VMEM((SEQ, D_M), jnp.float32),
                        pltpu.VMEM((SEQ, D_M), jnp.bfloat16)],
        compiler_params=pltpu.CompilerParams(
            dimension_semantics=("arbitrary",)),
    )(x2d, choice, w1, w2)
    return out[None]
